# baseline (speedup 1.0000x reference)
import jax
import jax.numpy as jnp
from jax import lax
from jax.experimental import pallas as pl
from jax.experimental.pallas import tpu as pltpu
from jax.experimental.pallas import tpu_sc as plsc

_SEP = 2
_B, _S, _D = 16, 2048, 1024
_HALF = _D // 2
_L = 16
_KV = _HALF // _L
_CH = 64
_NBUF = 2
_R0 = 768
_NCH_HEAD = _R0 // _CH
_BS = 256


def _sc_body(hid, ids, am, p1_out, p2_out, q_out,
             ids_v, am_v, acc, res1, res2, qv, buf0, buf1, sem0, sem1):
    bufs = [buf0, buf1]
    sems = [sem0, sem1]
    h = lax.axis_index("c")
    b = lax.axis_index("s")

    pltpu.sync_copy(ids.at[b], ids_v)
    pltpu.sync_copy(am.at[b], am_v)
    for i in range(_NBUF):
        pltpu.async_copy(
            hid.at[b, pl.ds(i * _CH, _CH), pl.ds(h * _HALF, _HALF)],
            bufs[i], sems[i])

    zeros = jnp.zeros((_L,), jnp.float32)
    for k in range(2 * _KV):
        acc[pl.ds(k * _L, _L)] = zeros

    lane = lax.iota(jnp.int32, _L)

    def pass1(j, carry):
        m1v, vlv = carry
        v = ids_v[pl.ds(j * _L, _L)]
        a = am_v[pl.ds(j * _L, _L)]
        pos = lane + j * _L
        cand = jnp.where(v == _SEP, pos, _S)
        return jnp.minimum(m1v, cand), vlv + a

    m1v, vlv = lax.fori_loop(
        0, _S // _L, pass1,
        (jnp.full((_L,), _S, jnp.int32), jnp.zeros((_L,), jnp.int32)))
    sep1 = m1v[0]
    valid_len = vlv[0]
    for i in range(1, _L):
        sep1 = jnp.minimum(sep1, m1v[i])
        valid_len = valid_len + vlv[i]

    def pass2(j, m2v):
        v = ids_v[pl.ds(j * _L, _L)]
        pos = lane + j * _L
        cand = jnp.where((v == _SEP) & (pos > sep1), pos, _S)
        return jnp.minimum(m2v, cand)

    m2v = lax.fori_loop(0, _S // _L, pass2, jnp.full((_L,), _S, jnp.int32))
    sep2 = m2v[0]
    for i in range(1, _L):
        sep2 = jnp.minimum(sep2, m2v[i])

    end_pos = jnp.minimum(valid_len - 1, _S)
    has_two = sep2 < _S
    fb_big = valid_len > 2
    fb_lo = jnp.where(fb_big, 1, 0)
    fb_hi = jnp.where(fb_big, valid_len - 1, 1)
    lo1 = jnp.where(has_two, 1, fb_lo)
    hi1 = jnp.where(has_two, sep1, fb_hi)
    lo2 = jnp.where(has_two, sep2 + 1, fb_lo)
    hi2 = jnp.where(has_two, end_pos, fb_hi)
    lo1 = jnp.minimum(lo1, hi1)
    lo2 = jnp.minimum(lo2, hi2)

    zero_accs = tuple(jnp.zeros((_L,), jnp.float32) for _ in range(_KV))

    def seg_accumulate(bufref, a, bnd, acc_off, negate=False):
        def row_body(j, accs):
            return tuple(accs[k] + bufref[j, pl.ds(k * _L, _L)]
                         for k in range(_KV))

        accs = lax.fori_loop(a, bnd, row_body, zero_accs)
        for k in range(_KV):
            v = jnp.negative(accs[k]) if negate else accs[k]
            plsc.addupdate(acc.at[pl.ds(acc_off + k * _L, _L)], v)

    def group_body(g, carry):
        for i in range(_NBUF):
            c = g * _NBUF + i
            base = c * _CH
            pltpu.make_async_copy(
                hid.at[b, pl.ds(0, _CH), pl.ds(h * _HALF, _HALF)],
                bufs[i], sems[i]).wait()

            a1 = jnp.clip(lo1 - base, 0, _CH)
            b1 = jnp.clip(hi1 - base, 0, _CH)
            a2 = jnp.clip(lo2 - base, 0, _CH)
            b2 = jnp.clip(hi2 - base, 0, _CH)
            seg_accumulate(bufs[i], a1, b1, 0)
            seg_accumulate(bufs[i], a2, b2, _HALF)

            nxt = c + _NBUF

            @pl.when(nxt < _NCH_HEAD)
            def _():
                pltpu.async_copy(
                    hid.at[b, pl.ds(nxt * _CH, _CH), pl.ds(h * _HALF, _HALF)],
                    bufs[i], sems[i])
        return carry

    lax.fori_loop(0, _NCH_HEAD // _NBUF, group_body, 0)

    u1 = hi1 > _R0
    u2 = hi2 > _R0
    b1_hi = jnp.clip(lo1, _R0, _S)
    b2_hi = jnp.clip(lo2, _R0, _S)
    c1_lo = jnp.where(u1, jnp.clip(hi1, _R0, _S), _S)
    c2_lo = jnp.where(u2, jnp.clip(hi2, _R0, _S), _S)

    def range_subtract(lo, hi, acc_off, bufref, sem):
        c0 = lax.shift_right_logical(lo, 6)
        c1 = lax.shift_right_logical(hi + (_CH - 1), 6)

        def chunk_body(ci, carry):
            start = ci * _CH
            pltpu.async_copy(
                hid.at[b, pl.ds(start, _CH), pl.ds(h * _HALF, _HALF)],
                bufref, sem).wait()
            a = jnp.clip(lo - start, 0, _CH)
            e = jnp.clip(hi - start, 0, _CH)
            seg_accumulate(bufref, a, e, acc_off, negate=True)
            return carry

        lax.fori_loop(c0, c1, chunk_body, 0)

    range_subtract(jnp.int32(_R0), b1_hi, 0, bufs[0], sems[0])
    range_subtract(c1_lo, jnp.int32(_S), 0, bufs[0], sems[0])
    range_subtract(jnp.int32(_R0), b2_hi, _HALF, bufs[1], sems[1])
    range_subtract(c2_lo, jnp.int32(_S), _HALF, bufs[1], sems[1])

    def _recip(x):
        xi = lax.bitcast_convert_type(x, jnp.int32)
        y = lax.bitcast_convert_type(jnp.int32(0x7EF311C3) - xi, jnp.float32)
        for _ in range(3):
            y = y * (2.0 - x * y)
        return y

    cnt1 = jnp.maximum(hi1 - lo1, 0)
    cnt2 = jnp.maximum(hi2 - lo2, 0)
    inv1 = _recip(jnp.maximum(cnt1, 1).astype(jnp.float32))
    inv2 = _recip(jnp.maximum(cnt2, 1).astype(jnp.float32))
    inv1v = jnp.full((_L,), inv1, jnp.float32)
    inv2v = jnp.full((_L,), inv2, jnp.float32)
    for k in range(_KV):
        res1[pl.ds(k * _L, _L)] = acc[pl.ds(k * _L, _L)] * inv1v
        res2[pl.ds(k * _L, _L)] = acc[pl.ds(_HALF + k * _L, _L)] * inv2v
    pltpu.sync_copy(res1, p1_out.at[b, pl.ds(h * _HALF, _HALF)])
    pltpu.sync_copy(res2, p2_out.at[b, pl.ds(h * _HALF, _HALF)])

    @pl.when(h == 0)
    def _():
        q1 = jnp.where(u1, inv1, 0.0)
        q2 = jnp.where(u2, inv2, 0.0)
        qv[pl.ds(0, _L)] = jnp.full((_L,), q1, jnp.float32)
        qv[pl.ds(_L, _L)] = jnp.full((_L,), q2, jnp.float32)
        pltpu.sync_copy(qv, q_out.at[b])


def _sc_call(hid, ids, am):
    f = pl.kernel(
        _sc_body,
        out_type=(jax.ShapeDtypeStruct((_B, _D), jnp.float32),
                  jax.ShapeDtypeStruct((_B, _D), jnp.float32),
                  jax.ShapeDtypeStruct((_B, 2 * _L), jnp.float32)),
        mesh=plsc.VectorSubcoreMesh(core_axis_name="c", subcore_axis_name="s",
                                    num_cores=2, num_subcores=16),
        scratch_types=[
            pltpu.VMEM((_S,), jnp.int32),
            pltpu.VMEM((_S,), jnp.int32),
            pltpu.VMEM((2 * _HALF,), jnp.float32),
            pltpu.VMEM((_HALF,), jnp.float32),
            pltpu.VMEM((_HALF,), jnp.float32),
            pltpu.VMEM((2 * _L,), jnp.float32),
            pltpu.VMEM((_CH, _HALF), jnp.float32),
            pltpu.VMEM((_CH, _HALF), jnp.float32),
            pltpu.SemaphoreType.DMA,
            pltpu.SemaphoreType.DMA,
        ],
    )
    return f(hid, ids, am)


def _tc_sum_body(x_ref, o_ref):
    j = pl.program_id(1)

    @pl.when(j == 0)
    def _():
        o_ref[...] = jnp.zeros_like(o_ref)

    o_ref[...] += jnp.sum(x_ref[...], axis=1, keepdims=True)


def _tc_sum(hid):
    out = pl.pallas_call(
        _tc_sum_body,
        grid=(_B, (_S - _R0) // _BS),
        in_specs=[pl.BlockSpec((1, _BS, _D),
                               lambda b, j: (b, _R0 // _BS + j, 0))],
        out_specs=pl.BlockSpec((1, 1, _D), lambda b, j: (b, 0, 0)),
        out_shape=jax.ShapeDtypeStruct((_B, 1, _D), jnp.float32),
        compiler_params=pltpu.CompilerParams(
            dimension_semantics=("parallel", "arbitrary")),
    )(hid)
    return out.reshape(_B, _D)


def _combine_body(t_ref, p1_ref, p2_ref, q_ref, o1_ref, o2_ref):
    t = t_ref[...]
    q1 = q_ref[:, 0:1]
    q2 = q_ref[:, _L:_L + 1]
    o1_ref[...] = p1_ref[...] + q1 * t
    o2_ref[...] = p2_ref[...] + q2 * t


def _combine(t, p1, p2, q):
    return pl.pallas_call(
        _combine_body,
        out_shape=(jax.ShapeDtypeStruct((_B, _D), jnp.float32),
                   jax.ShapeDtypeStruct((_B, _D), jnp.float32)),
    )(t, p1, p2, q)


@jax.jit
def kernel(hidden_states, input_ids, attention_mask):
    t = _tc_sum(hidden_states)
    p1, p2, q = _sc_call(hidden_states,
                         input_ids.astype(jnp.int32),
                         attention_mask.astype(jnp.int32))
    return _combine(t, p1, p2, q)


# speedup vs baseline: 1.2033x; 1.2033x over previous
import jax
import jax.numpy as jnp
from jax import lax
from jax.experimental import pallas as pl
from jax.experimental.pallas import tpu as pltpu
from jax.experimental.pallas import tpu_sc as plsc

_SEP = 2
_B, _S, _D = 16, 2048, 1024
_HALF = _D // 2
_L = 16
_KV = _HALF // _L
_CH = 64
_NBUF = 2
_R0 = 512
_NCH_HEAD = _R0 // _CH
_BS = 512
_NTB = (_S - _R0) // _BS


def _sc_body(hid, ids, am, p1_out, p2_out, q_out,
             ids_v, am_v, acc, res1, res2, qv, buf0, buf1, sem0, sem1):
    bufs = [buf0, buf1]
    sems = [sem0, sem1]
    h = lax.axis_index("c")
    b = lax.axis_index("s")

    pltpu.sync_copy(ids.at[b], ids_v)
    pltpu.sync_copy(am.at[b], am_v)
    for i in range(_NBUF):
        pltpu.async_copy(
            hid.at[b, pl.ds(i * _CH, _CH), pl.ds(h * _HALF, _HALF)],
            bufs[i], sems[i])

    zeros = jnp.zeros((_L,), jnp.float32)
    for k in range(2 * _KV):
        acc[pl.ds(k * _L, _L)] = zeros

    lane = lax.iota(jnp.int32, _L)

    def pass1(j, carry):
        m1v, vlv = carry
        v = ids_v[pl.ds(j * _L, _L)]
        a = am_v[pl.ds(j * _L, _L)]
        pos = lane + j * _L
        cand = jnp.where(v == _SEP, pos, _S)
        return jnp.minimum(m1v, cand), vlv + a

    m1v, vlv = lax.fori_loop(
        0, _S // _L, pass1,
        (jnp.full((_L,), _S, jnp.int32), jnp.zeros((_L,), jnp.int32)))
    sep1 = m1v[0]
    valid_len = vlv[0]
    for i in range(1, _L):
        sep1 = jnp.minimum(sep1, m1v[i])
        valid_len = valid_len + vlv[i]

    def pass2(j, m2v):
        v = ids_v[pl.ds(j * _L, _L)]
        pos = lane + j * _L
        cand = jnp.where((v == _SEP) & (pos > sep1), pos, _S)
        return jnp.minimum(m2v, cand)

    m2v = lax.fori_loop(0, _S // _L, pass2, jnp.full((_L,), _S, jnp.int32))
    sep2 = m2v[0]
    for i in range(1, _L):
        sep2 = jnp.minimum(sep2, m2v[i])

    end_pos = jnp.minimum(valid_len - 1, _S)
    has_two = sep2 < _S
    fb_big = valid_len > 2
    fb_lo = jnp.where(fb_big, 1, 0)
    fb_hi = jnp.where(fb_big, valid_len - 1, 1)
    lo1 = jnp.where(has_two, 1, fb_lo)
    hi1 = jnp.where(has_two, sep1, fb_hi)
    lo2 = jnp.where(has_two, sep2 + 1, fb_lo)
    hi2 = jnp.where(has_two, end_pos, fb_hi)
    lo1 = jnp.minimum(lo1, hi1)
    lo2 = jnp.minimum(lo2, hi2)

    zero_accs = tuple(jnp.zeros((_L,), jnp.float32) for _ in range(_KV))

    def seg_accumulate(bufref, a, bnd, acc_off, negate=False):
        def row_body(j, accs):
            return tuple(accs[k] + bufref[j, pl.ds(k * _L, _L)]
                         for k in range(_KV))

        accs = lax.fori_loop(a, bnd, row_body, zero_accs)
        for k in range(_KV):
            v = jnp.negative(accs[k]) if negate else accs[k]
            plsc.addupdate(acc.at[pl.ds(acc_off + k * _L, _L)], v)

    def group_body(g, carry):
        for i in range(_NBUF):
            c = g * _NBUF + i
            base = c * _CH
            pltpu.make_async_copy(
                hid.at[b, pl.ds(0, _CH), pl.ds(h * _HALF, _HALF)],
                bufs[i], sems[i]).wait()

            a1 = jnp.clip(lo1 - base, 0, _CH)
            b1 = jnp.clip(hi1 - base, 0, _CH)
            a2 = jnp.clip(lo2 - base, 0, _CH)
            b2 = jnp.clip(hi2 - base, 0, _CH)
            seg_accumulate(bufs[i], a1, b1, 0)
            seg_accumulate(bufs[i], a2, b2, _HALF)

            nxt = c + _NBUF

            @pl.when(nxt < _NCH_HEAD)
            def _():
                pltpu.async_copy(
                    hid.at[b, pl.ds(nxt * _CH, _CH), pl.ds(h * _HALF, _HALF)],
                    bufs[i], sems[i])
        return carry

    lax.fori_loop(0, _NCH_HEAD // _NBUF, group_body, 0)

    u1 = hi1 > _R0
    u2 = hi2 > _R0
    b1_hi = jnp.clip(lo1, _R0, _S)
    b2_hi = jnp.clip(lo2, _R0, _S)
    c1_lo = jnp.where(u1, jnp.clip(hi1, _R0, _S), _S)
    c2_lo = jnp.where(u2, jnp.clip(hi2, _R0, _S), _S)

    def range_subtract(lo, hi, acc_off, bufref, sem):
        c0 = lax.shift_right_logical(lo, 6)
        c1 = lax.shift_right_logical(hi + (_CH - 1), 6)

        def chunk_body(ci, carry):
            start = ci * _CH
            pltpu.async_copy(
                hid.at[b, pl.ds(start, _CH), pl.ds(h * _HALF, _HALF)],
                bufref, sem).wait()
            a = jnp.clip(lo - start, 0, _CH)
            e = jnp.clip(hi - start, 0, _CH)
            seg_accumulate(bufref, a, e, acc_off, negate=True)
            return carry

        lax.fori_loop(c0, c1, chunk_body, 0)

    range_subtract(jnp.int32(_R0), b1_hi, 0, bufs[0], sems[0])
    range_subtract(c1_lo, jnp.int32(_S), 0, bufs[0], sems[0])
    range_subtract(jnp.int32(_R0), b2_hi, _HALF, bufs[1], sems[1])
    range_subtract(c2_lo, jnp.int32(_S), _HALF, bufs[1], sems[1])

    def _recip(x):
        xi = lax.bitcast_convert_type(x, jnp.int32)
        y = lax.bitcast_convert_type(jnp.int32(0x7EF311C3) - xi, jnp.float32)
        for _ in range(3):
            y = y * (2.0 - x * y)
        return y

    cnt1 = jnp.maximum(hi1 - lo1, 0)
    cnt2 = jnp.maximum(hi2 - lo2, 0)
    inv1 = _recip(jnp.maximum(cnt1, 1).astype(jnp.float32))
    inv2 = _recip(jnp.maximum(cnt2, 1).astype(jnp.float32))
    inv1v = jnp.full((_L,), inv1, jnp.float32)
    inv2v = jnp.full((_L,), inv2, jnp.float32)
    for k in range(_KV):
        res1[pl.ds(k * _L, _L)] = acc[pl.ds(k * _L, _L)] * inv1v
        res2[pl.ds(k * _L, _L)] = acc[pl.ds(_HALF + k * _L, _L)] * inv2v
    pltpu.sync_copy(res1, p1_out.at[b, pl.ds(h * _HALF, _HALF)])
    pltpu.sync_copy(res2, p2_out.at[b, pl.ds(h * _HALF, _HALF)])

    @pl.when(h == 0)
    def _():
        q1 = jnp.where(u1, inv1, 0.0)
        q2 = jnp.where(u2, inv2, 0.0)
        qv[pl.ds(0, _L)] = jnp.full((_L,), q1, jnp.float32)
        qv[pl.ds(_L, _L)] = jnp.full((_L,), q2, jnp.float32)
        pltpu.sync_copy(qv, q_out.at[b])


def _sc_call(hid, ids, am):
    f = pl.kernel(
        _sc_body,
        out_type=(jax.ShapeDtypeStruct((_B, _D), jnp.float32),
                  jax.ShapeDtypeStruct((_B, _D), jnp.float32),
                  jax.ShapeDtypeStruct((_B, 2 * _L), jnp.float32)),
        mesh=plsc.VectorSubcoreMesh(core_axis_name="c", subcore_axis_name="s",
                                    num_cores=2, num_subcores=16),
        scratch_types=[
            pltpu.VMEM((_S,), jnp.int32),
            pltpu.VMEM((_S,), jnp.int32),
            pltpu.VMEM((2 * _HALF,), jnp.float32),
            pltpu.VMEM((_HALF,), jnp.float32),
            pltpu.VMEM((_HALF,), jnp.float32),
            pltpu.VMEM((2 * _L,), jnp.float32),
            pltpu.VMEM((_CH, _HALF), jnp.float32),
            pltpu.VMEM((_CH, _HALF), jnp.float32),
            pltpu.SemaphoreType.DMA,
            pltpu.SemaphoreType.DMA,
        ],
    )
    return f(hid, ids, am)


def _tc_sum_body(x_ref, o_ref):
    o_ref[...] = jnp.sum(x_ref[...], axis=1,
                         keepdims=True).reshape(o_ref.shape)


def _tc_sum(hid):
    out = pl.pallas_call(
        _tc_sum_body,
        grid=(_B, _NTB),
        in_specs=[pl.BlockSpec((1, _BS, _D),
                               lambda b, j: (b, _R0 // _BS + j, 0))],
        out_specs=pl.BlockSpec((1, 1, 1, _D), lambda b, j: (b, j, 0, 0)),
        out_shape=jax.ShapeDtypeStruct((_B, _NTB, 1, _D), jnp.float32),
        compiler_params=pltpu.CompilerParams(
            dimension_semantics=("parallel", "arbitrary")),
    )(hid)
    return out.reshape(_B, _NTB, _D)


def _combine_body(t_ref, p1_ref, p2_ref, q_ref, o1_ref, o2_ref):
    t = jnp.sum(t_ref[...], axis=1)
    q1 = q_ref[:, 0:1]
    q2 = q_ref[:, _L:_L + 1]
    o1_ref[...] = p1_ref[...] + q1 * t
    o2_ref[...] = p2_ref[...] + q2 * t


def _combine(t, p1, p2, q):
    return pl.pallas_call(
        _combine_body,
        out_shape=(jax.ShapeDtypeStruct((_B, _D), jnp.float32),
                   jax.ShapeDtypeStruct((_B, _D), jnp.float32)),
    )(t, p1, p2, q)


@jax.jit
def kernel(hidden_states, input_ids, attention_mask):
    t = _tc_sum(hidden_states)
    p1, p2, q = _sc_call(hidden_states,
                         input_ids.astype(jnp.int32),
                         attention_mask.astype(jnp.int32))
    return _combine(t, p1, p2, q)


# speedup vs baseline: 1.3473x; 1.1197x over previous
import jax
import jax.numpy as jnp
from jax import lax
from jax.experimental import pallas as pl
from jax.experimental.pallas import tpu as pltpu
from jax.experimental.pallas import tpu_sc as plsc

_SEP = 2
_B, _S, _D = 16, 2048, 1024
_HALF = _D // 2
_L = 16
_KV = _HALF // _L
_CH = 64
_NBUF = 2
_R0 = 512
_NCH_HEAD = _R0 // _CH
_BS = 512
_NTB = (_S - _R0) // _BS
_BB = 4


def _sc_body(hid, ids, am, p1_out, p2_out, q_out,
             ids_v, am_v, acc, res1, res2, qv, buf0, buf1, sem0, sem1):
    bufs = [buf0, buf1]
    sems = [sem0, sem1]
    h = lax.axis_index("c")
    b = lax.axis_index("s")

    pltpu.sync_copy(ids.at[b], ids_v)
    pltpu.sync_copy(am.at[b], am_v)
    for i in range(_NBUF):
        pltpu.async_copy(
            hid.at[b, pl.ds(i * _CH, _CH), pl.ds(h * _HALF, _HALF)],
            bufs[i], sems[i])

    zeros = jnp.zeros((_L,), jnp.float32)
    for k in range(2 * _KV):
        acc[pl.ds(k * _L, _L)] = zeros

    lane = lax.iota(jnp.int32, _L)

    def pass1(j, carry):
        m1v, vlv = carry
        v = ids_v[pl.ds(j * _L, _L)]
        a = am_v[pl.ds(j * _L, _L)]
        pos = lane + j * _L
        cand = jnp.where(v == _SEP, pos, _S)
        return jnp.minimum(m1v, cand), vlv + a

    m1v, vlv = lax.fori_loop(
        0, _S // _L, pass1,
        (jnp.full((_L,), _S, jnp.int32), jnp.zeros((_L,), jnp.int32)))
    sep1 = m1v[0]
    valid_len = vlv[0]
    for i in range(1, _L):
        sep1 = jnp.minimum(sep1, m1v[i])
        valid_len = valid_len + vlv[i]

    def pass2(j, m2v):
        v = ids_v[pl.ds(j * _L, _L)]
        pos = lane + j * _L
        cand = jnp.where((v == _SEP) & (pos > sep1), pos, _S)
        return jnp.minimum(m2v, cand)

    m2v = lax.fori_loop(0, _S // _L, pass2, jnp.full((_L,), _S, jnp.int32))
    sep2 = m2v[0]
    for i in range(1, _L):
        sep2 = jnp.minimum(sep2, m2v[i])

    end_pos = jnp.minimum(valid_len - 1, _S)
    has_two = sep2 < _S
    fb_big = valid_len > 2
    fb_lo = jnp.where(fb_big, 1, 0)
    fb_hi = jnp.where(fb_big, valid_len - 1, 1)
    lo1 = jnp.where(has_two, 1, fb_lo)
    hi1 = jnp.where(has_two, sep1, fb_hi)
    lo2 = jnp.where(has_two, sep2 + 1, fb_lo)
    hi2 = jnp.where(has_two, end_pos, fb_hi)
    lo1 = jnp.minimum(lo1, hi1)
    lo2 = jnp.minimum(lo2, hi2)

    zero_accs = tuple(jnp.zeros((_L,), jnp.float32) for _ in range(_KV))

    def seg_accumulate(bufref, a, bnd, acc_off, negate=False):
        def row_body(j, accs):
            return tuple(accs[k] + bufref[j, pl.ds(k * _L, _L)]
                         for k in range(_KV))

        accs = lax.fori_loop(a, bnd, row_body, zero_accs)
        for k in range(_KV):
            v = jnp.negative(accs[k]) if negate else accs[k]
            plsc.addupdate(acc.at[pl.ds(acc_off + k * _L, _L)], v)

    def group_body(g, carry):
        for i in range(_NBUF):
            c = g * _NBUF + i
            base = c * _CH
            pltpu.make_async_copy(
                hid.at[b, pl.ds(0, _CH), pl.ds(h * _HALF, _HALF)],
                bufs[i], sems[i]).wait()

            a1 = jnp.clip(lo1 - base, 0, _CH)
            b1 = jnp.clip(hi1 - base, 0, _CH)
            a2 = jnp.clip(lo2 - base, 0, _CH)
            b2 = jnp.clip(hi2 - base, 0, _CH)
            seg_accumulate(bufs[i], a1, b1, 0)
            seg_accumulate(bufs[i], a2, b2, _HALF)

            nxt = c + _NBUF

            @pl.when(nxt < _NCH_HEAD)
            def _():
                pltpu.async_copy(
                    hid.at[b, pl.ds(nxt * _CH, _CH), pl.ds(h * _HALF, _HALF)],
                    bufs[i], sems[i])
        return carry

    lax.fori_loop(0, _NCH_HEAD // _NBUF, group_body, 0)

    u1 = hi1 > _R0
    u2 = hi2 > _R0
    b1_hi = jnp.clip(lo1, _R0, _S)
    b2_hi = jnp.clip(lo2, _R0, _S)
    c1_lo = jnp.where(u1, jnp.clip(hi1, _R0, _S), _S)
    c2_lo = jnp.where(u2, jnp.clip(hi2, _R0, _S), _S)

    def range_subtract(lo, hi, acc_off, bufref, sem):
        c0 = lax.shift_right_logical(lo, 6)
        c1 = lax.shift_right_logical(hi + (_CH - 1), 6)

        def chunk_body(ci, carry):
            start = ci * _CH
            pltpu.async_copy(
                hid.at[b, pl.ds(start, _CH), pl.ds(h * _HALF, _HALF)],
                bufref, sem).wait()
            a = jnp.clip(lo - start, 0, _CH)
            e = jnp.clip(hi - start, 0, _CH)
            seg_accumulate(bufref, a, e, acc_off, negate=True)
            return carry

        lax.fori_loop(c0, c1, chunk_body, 0)

    range_subtract(jnp.int32(_R0), b1_hi, 0, bufs[0], sems[0])
    range_subtract(c1_lo, jnp.int32(_S), 0, bufs[0], sems[0])
    range_subtract(jnp.int32(_R0), b2_hi, _HALF, bufs[1], sems[1])
    range_subtract(c2_lo, jnp.int32(_S), _HALF, bufs[1], sems[1])

    def _recip(x):
        xi = lax.bitcast_convert_type(x, jnp.int32)
        y = lax.bitcast_convert_type(jnp.int32(0x7EF311C3) - xi, jnp.float32)
        for _ in range(3):
            y = y * (2.0 - x * y)
        return y

    cnt1 = jnp.maximum(hi1 - lo1, 0)
    cnt2 = jnp.maximum(hi2 - lo2, 0)
    inv1 = _recip(jnp.maximum(cnt1, 1).astype(jnp.float32))
    inv2 = _recip(jnp.maximum(cnt2, 1).astype(jnp.float32))
    inv1v = jnp.full((_L,), inv1, jnp.float32)
    inv2v = jnp.full((_L,), inv2, jnp.float32)
    for k in range(_KV):
        res1[pl.ds(k * _L, _L)] = acc[pl.ds(k * _L, _L)] * inv1v
        res2[pl.ds(k * _L, _L)] = acc[pl.ds(_HALF + k * _L, _L)] * inv2v
    pltpu.sync_copy(res1, p1_out.at[b, pl.ds(h * _HALF, _HALF)])
    pltpu.sync_copy(res2, p2_out.at[b, pl.ds(h * _HALF, _HALF)])

    @pl.when(h == 0)
    def _():
        q1 = jnp.where(u1, inv1, 0.0)
        q2 = jnp.where(u2, inv2, 0.0)
        qv[pl.ds(0, _L)] = jnp.full((_L,), q1, jnp.float32)
        qv[pl.ds(_L, _L)] = jnp.full((_L,), q2, jnp.float32)
        pltpu.sync_copy(qv, q_out.at[b])


def _sc_call(hid, ids, am):
    f = pl.kernel(
        _sc_body,
        out_type=(jax.ShapeDtypeStruct((_B, _D), jnp.float32),
                  jax.ShapeDtypeStruct((_B, _D), jnp.float32),
                  jax.ShapeDtypeStruct((_B, 2 * _L), jnp.float32)),
        mesh=plsc.VectorSubcoreMesh(core_axis_name="c", subcore_axis_name="s",
                                    num_cores=2, num_subcores=16),
        scratch_types=[
            pltpu.VMEM((_S,), jnp.int32),
            pltpu.VMEM((_S,), jnp.int32),
            pltpu.VMEM((2 * _HALF,), jnp.float32),
            pltpu.VMEM((_HALF,), jnp.float32),
            pltpu.VMEM((_HALF,), jnp.float32),
            pltpu.VMEM((2 * _L,), jnp.float32),
            pltpu.VMEM((_CH, _HALF), jnp.float32),
            pltpu.VMEM((_CH, _HALF), jnp.float32),
            pltpu.SemaphoreType.DMA,
            pltpu.SemaphoreType.DMA,
        ],
    )
    return f(hid, ids, am)


def _tc_sum_body(x_ref, o_ref):
    o_ref[...] = jnp.sum(x_ref[...], axis=1,
                         keepdims=True).reshape(o_ref.shape)


def _tc_sum(hid):
    out = pl.pallas_call(
        _tc_sum_body,
        grid=(_B // _BB, _NTB),
        in_specs=[pl.BlockSpec((_BB, _BS, _D),
                               lambda b, j: (b, _R0 // _BS + j, 0))],
        out_specs=pl.BlockSpec((1, 1, _BB, _D), lambda b, j: (b, j, 0, 0)),
        out_shape=jax.ShapeDtypeStruct((_B // _BB, _NTB, _BB, _D),
                                       jnp.float32),
        compiler_params=pltpu.CompilerParams(
            dimension_semantics=("parallel", "arbitrary")),
    )(hid)
    return out


def _combine_body(t_ref, p1_ref, p2_ref, q_ref, o1_ref, o2_ref):
    t = jnp.sum(t_ref[...], axis=1).reshape(_B, _D)
    q1 = q_ref[:, 0:1]
    q2 = q_ref[:, _L:_L + 1]
    o1_ref[...] = p1_ref[...] + q1 * t
    o2_ref[...] = p2_ref[...] + q2 * t


def _combine(t, p1, p2, q):
    return pl.pallas_call(
        _combine_body,
        out_shape=(jax.ShapeDtypeStruct((_B, _D), jnp.float32),
                   jax.ShapeDtypeStruct((_B, _D), jnp.float32)),
    )(t, p1, p2, q)


@jax.jit
def kernel(hidden_states, input_ids, attention_mask):
    t = _tc_sum(hidden_states)
    p1, p2, q = _sc_call(hidden_states,
                         input_ids.astype(jnp.int32),
                         attention_mask.astype(jnp.int32))
    return _combine(t, p1, p2, q)


# speedup vs baseline: 1.3785x; 1.0232x over previous
import jax
import jax.numpy as jnp
from jax import lax
from jax.experimental import pallas as pl
from jax.experimental.pallas import tpu as pltpu
from jax.experimental.pallas import tpu_sc as plsc

_SEP = 2
_B, _S, _D = 16, 2048, 1024
_HALF = _D // 2
_L = 16
_KV = _HALF // _L
_CH = 64
_NBUF = 2
_R0 = 512
_NCH_HEAD = _R0 // _CH
_BS = 512
_NTB = (_S - _R0) // _BS
_BB = 8


def _sc_body(hid, ids, am, p1_out, p2_out, q_out,
             ids_v, am_v, acc, res1, res2, qv, buf0, buf1, sem0, sem1):
    bufs = [buf0, buf1]
    sems = [sem0, sem1]
    h = lax.axis_index("c")
    b = lax.axis_index("s")

    pltpu.sync_copy(ids.at[b], ids_v)
    pltpu.sync_copy(am.at[b], am_v)
    for i in range(_NBUF):
        pltpu.async_copy(
            hid.at[b, pl.ds(i * _CH, _CH), pl.ds(h * _HALF, _HALF)],
            bufs[i], sems[i])

    zeros = jnp.zeros((_L,), jnp.float32)
    for k in range(2 * _KV):
        acc[pl.ds(k * _L, _L)] = zeros

    lane = lax.iota(jnp.int32, _L)

    def pass1(j, carry):
        m1v, vlv = carry
        v = ids_v[pl.ds(j * _L, _L)]
        a = am_v[pl.ds(j * _L, _L)]
        pos = lane + j * _L
        cand = jnp.where(v == _SEP, pos, _S)
        return jnp.minimum(m1v, cand), vlv + a

    m1v, vlv = lax.fori_loop(
        0, _S // _L, pass1,
        (jnp.full((_L,), _S, jnp.int32), jnp.zeros((_L,), jnp.int32)))
    sep1 = m1v[0]
    valid_len = vlv[0]
    for i in range(1, _L):
        sep1 = jnp.minimum(sep1, m1v[i])
        valid_len = valid_len + vlv[i]

    def pass2(j, m2v):
        v = ids_v[pl.ds(j * _L, _L)]
        pos = lane + j * _L
        cand = jnp.where((v == _SEP) & (pos > sep1), pos, _S)
        return jnp.minimum(m2v, cand)

    m2v = lax.fori_loop(0, _S // _L, pass2, jnp.full((_L,), _S, jnp.int32))
    sep2 = m2v[0]
    for i in range(1, _L):
        sep2 = jnp.minimum(sep2, m2v[i])

    end_pos = jnp.minimum(valid_len - 1, _S)
    has_two = sep2 < _S
    fb_big = valid_len > 2
    fb_lo = jnp.where(fb_big, 1, 0)
    fb_hi = jnp.where(fb_big, valid_len - 1, 1)
    lo1 = jnp.where(has_two, 1, fb_lo)
    hi1 = jnp.where(has_two, sep1, fb_hi)
    lo2 = jnp.where(has_two, sep2 + 1, fb_lo)
    hi2 = jnp.where(has_two, end_pos, fb_hi)
    lo1 = jnp.minimum(lo1, hi1)
    lo2 = jnp.minimum(lo2, hi2)

    zero_accs = tuple(jnp.zeros((_L,), jnp.float32) for _ in range(_KV))

    def seg_accumulate(bufref, a, bnd, acc_off, negate=False):
        def row_body(j, accs):
            return tuple(accs[k] + bufref[j, pl.ds(k * _L, _L)]
                         for k in range(_KV))

        accs = lax.fori_loop(a, bnd, row_body, zero_accs)
        for k in range(_KV):
            v = jnp.negative(accs[k]) if negate else accs[k]
            plsc.addupdate(acc.at[pl.ds(acc_off + k * _L, _L)], v)

    def group_body(g, carry):
        for i in range(_NBUF):
            c = g * _NBUF + i
            base = c * _CH
            pltpu.make_async_copy(
                hid.at[b, pl.ds(0, _CH), pl.ds(h * _HALF, _HALF)],
                bufs[i], sems[i]).wait()

            a1 = jnp.clip(lo1 - base, 0, _CH)
            b1 = jnp.clip(hi1 - base, 0, _CH)
            a2 = jnp.clip(lo2 - base, 0, _CH)
            b2 = jnp.clip(hi2 - base, 0, _CH)
            seg_accumulate(bufs[i], a1, b1, 0)
            seg_accumulate(bufs[i], a2, b2, _HALF)

            nxt = c + _NBUF

            @pl.when(nxt < _NCH_HEAD)
            def _():
                pltpu.async_copy(
                    hid.at[b, pl.ds(nxt * _CH, _CH), pl.ds(h * _HALF, _HALF)],
                    bufs[i], sems[i])
        return carry

    lax.fori_loop(0, _NCH_HEAD // _NBUF, group_body, 0)

    u1 = hi1 > _R0
    u2 = hi2 > _R0
    b1_hi = jnp.clip(lo1, _R0, _S)
    b2_hi = jnp.clip(lo2, _R0, _S)
    c1_lo = jnp.where(u1, jnp.clip(hi1, _R0, _S), _S)
    c2_lo = jnp.where(u2, jnp.clip(hi2, _R0, _S), _S)

    def range_subtract(lo, hi, acc_off, bufref, sem):
        c0 = lax.shift_right_logical(lo, 6)
        c1 = lax.shift_right_logical(hi + (_CH - 1), 6)

        def chunk_body(ci, carry):
            start = ci * _CH
            pltpu.async_copy(
                hid.at[b, pl.ds(start, _CH), pl.ds(h * _HALF, _HALF)],
                bufref, sem).wait()
            a = jnp.clip(lo - start, 0, _CH)
            e = jnp.clip(hi - start, 0, _CH)
            seg_accumulate(bufref, a, e, acc_off, negate=True)
            return carry

        lax.fori_loop(c0, c1, chunk_body, 0)

    range_subtract(jnp.int32(_R0), b1_hi, 0, bufs[0], sems[0])
    range_subtract(c1_lo, jnp.int32(_S), 0, bufs[0], sems[0])
    range_subtract(jnp.int32(_R0), b2_hi, _HALF, bufs[1], sems[1])
    range_subtract(c2_lo, jnp.int32(_S), _HALF, bufs[1], sems[1])

    def _recip(x):
        xi = lax.bitcast_convert_type(x, jnp.int32)
        y = lax.bitcast_convert_type(jnp.int32(0x7EF311C3) - xi, jnp.float32)
        for _ in range(3):
            y = y * (2.0 - x * y)
        return y

    cnt1 = jnp.maximum(hi1 - lo1, 0)
    cnt2 = jnp.maximum(hi2 - lo2, 0)
    inv1 = _recip(jnp.maximum(cnt1, 1).astype(jnp.float32))
    inv2 = _recip(jnp.maximum(cnt2, 1).astype(jnp.float32))
    inv1v = jnp.full((_L,), inv1, jnp.float32)
    inv2v = jnp.full((_L,), inv2, jnp.float32)
    for k in range(_KV):
        res1[pl.ds(k * _L, _L)] = acc[pl.ds(k * _L, _L)] * inv1v
        res2[pl.ds(k * _L, _L)] = acc[pl.ds(_HALF + k * _L, _L)] * inv2v
    pltpu.sync_copy(res1, p1_out.at[b, pl.ds(h * _HALF, _HALF)])
    pltpu.sync_copy(res2, p2_out.at[b, pl.ds(h * _HALF, _HALF)])

    @pl.when(h == 0)
    def _():
        q1 = jnp.where(u1, inv1, 0.0)
        q2 = jnp.where(u2, inv2, 0.0)
        qv[pl.ds(0, _L)] = jnp.full((_L,), q1, jnp.float32)
        qv[pl.ds(_L, _L)] = jnp.full((_L,), q2, jnp.float32)
        pltpu.sync_copy(qv, q_out.at[b])


def _sc_call(hid, ids, am):
    f = pl.kernel(
        _sc_body,
        out_type=(jax.ShapeDtypeStruct((_B, _D), jnp.float32),
                  jax.ShapeDtypeStruct((_B, _D), jnp.float32),
                  jax.ShapeDtypeStruct((_B, 2 * _L), jnp.float32)),
        mesh=plsc.VectorSubcoreMesh(core_axis_name="c", subcore_axis_name="s",
                                    num_cores=2, num_subcores=16),
        scratch_types=[
            pltpu.VMEM((_S,), jnp.int32),
            pltpu.VMEM((_S,), jnp.int32),
            pltpu.VMEM((2 * _HALF,), jnp.float32),
            pltpu.VMEM((_HALF,), jnp.float32),
            pltpu.VMEM((_HALF,), jnp.float32),
            pltpu.VMEM((2 * _L,), jnp.float32),
            pltpu.VMEM((_CH, _HALF), jnp.float32),
            pltpu.VMEM((_CH, _HALF), jnp.float32),
            pltpu.SemaphoreType.DMA,
            pltpu.SemaphoreType.DMA,
        ],
    )
    return f(hid, ids, am)


def _tc_sum_body(x_ref, o_ref):
    o_ref[...] = jnp.sum(x_ref[...], axis=1,
                         keepdims=True).reshape(o_ref.shape)


def _tc_sum(hid):
    out = pl.pallas_call(
        _tc_sum_body,
        grid=(_B // _BB, _NTB),
        in_specs=[pl.BlockSpec((_BB, _BS, _D),
                               lambda b, j: (b, _R0 // _BS + j, 0))],
        out_specs=pl.BlockSpec((1, 1, _BB, _D), lambda b, j: (b, j, 0, 0)),
        out_shape=jax.ShapeDtypeStruct((_B // _BB, _NTB, _BB, _D),
                                       jnp.float32),
        compiler_params=pltpu.CompilerParams(
            dimension_semantics=("parallel", "arbitrary")),
    )(hid)
    return out


def _combine_body(t_ref, p1_ref, p2_ref, q_ref, o1_ref, o2_ref):
    t = jnp.sum(t_ref[...], axis=1).reshape(_B, _D)
    q1 = q_ref[:, 0:1]
    q2 = q_ref[:, _L:_L + 1]
    o1_ref[...] = p1_ref[...] + q1 * t
    o2_ref[...] = p2_ref[...] + q2 * t


def _combine(t, p1, p2, q):
    return pl.pallas_call(
        _combine_body,
        out_shape=(jax.ShapeDtypeStruct((_B, _D), jnp.float32),
                   jax.ShapeDtypeStruct((_B, _D), jnp.float32)),
    )(t, p1, p2, q)


@jax.jit
def kernel(hidden_states, input_ids, attention_mask):
    t = _tc_sum(hidden_states)
    p1, p2, q = _sc_call(hidden_states,
                         input_ids.astype(jnp.int32),
                         attention_mask.astype(jnp.int32))
    return _combine(t, p1, p2, q)


# speedup vs baseline: 1.4471x; 1.0498x over previous
import jax
import jax.numpy as jnp
from jax import lax
from jax.experimental import pallas as pl
from jax.experimental.pallas import tpu as pltpu
from jax.experimental.pallas import tpu_sc as plsc

_SEP = 2
_B, _S, _D = 16, 2048, 1024
_HALF = _D // 2
_L = 16
_KV = _HALF // _L
_CH = 64
_NBUF = 2
_R0 = 512
_NCH_HEAD = _R0 // _CH
_BS = 512
_NTB = (_S - _R0) // _BS
_BB = 4
_NCS = 2
_CSW = _D // _NCS


def _sc_body(hid, ids, am, p1_out, p2_out, q_out,
             ids_v, am_v, acc, res1, res2, qv, buf0, buf1, sem0, sem1):
    bufs = [buf0, buf1]
    sems = [sem0, sem1]
    h = lax.axis_index("c")
    b = lax.axis_index("s")

    pltpu.sync_copy(ids.at[b], ids_v)
    pltpu.sync_copy(am.at[b], am_v)
    for i in range(_NBUF):
        pltpu.async_copy(
            hid.at[b, pl.ds(i * _CH, _CH), pl.ds(h * _HALF, _HALF)],
            bufs[i], sems[i])

    zeros = jnp.zeros((_L,), jnp.float32)
    for k in range(2 * _KV):
        acc[pl.ds(k * _L, _L)] = zeros

    lane = lax.iota(jnp.int32, _L)

    def pass1(j, carry):
        m1v, vlv = carry
        v = ids_v[pl.ds(j * _L, _L)]
        a = am_v[pl.ds(j * _L, _L)]
        pos = lane + j * _L
        cand = jnp.where(v == _SEP, pos, _S)
        return jnp.minimum(m1v, cand), vlv + a

    m1v, vlv = lax.fori_loop(
        0, _S // _L, pass1,
        (jnp.full((_L,), _S, jnp.int32), jnp.zeros((_L,), jnp.int32)))
    sep1 = m1v[0]
    valid_len = vlv[0]
    for i in range(1, _L):
        sep1 = jnp.minimum(sep1, m1v[i])
        valid_len = valid_len + vlv[i]

    def pass2(j, m2v):
        v = ids_v[pl.ds(j * _L, _L)]
        pos = lane + j * _L
        cand = jnp.where((v == _SEP) & (pos > sep1), pos, _S)
        return jnp.minimum(m2v, cand)

    m2v = lax.fori_loop(0, _S // _L, pass2, jnp.full((_L,), _S, jnp.int32))
    sep2 = m2v[0]
    for i in range(1, _L):
        sep2 = jnp.minimum(sep2, m2v[i])

    end_pos = jnp.minimum(valid_len - 1, _S)
    has_two = sep2 < _S
    fb_big = valid_len > 2
    fb_lo = jnp.where(fb_big, 1, 0)
    fb_hi = jnp.where(fb_big, valid_len - 1, 1)
    lo1 = jnp.where(has_two, 1, fb_lo)
    hi1 = jnp.where(has_two, sep1, fb_hi)
    lo2 = jnp.where(has_two, sep2 + 1, fb_lo)
    hi2 = jnp.where(has_two, end_pos, fb_hi)
    lo1 = jnp.minimum(lo1, hi1)
    lo2 = jnp.minimum(lo2, hi2)

    zero_accs = tuple(jnp.zeros((_L,), jnp.float32) for _ in range(_KV))

    def seg_accumulate(bufref, a, bnd, acc_off, negate=False):
        def row_body(j, accs):
            return tuple(accs[k] + bufref[j, pl.ds(k * _L, _L)]
                         for k in range(_KV))

        accs = lax.fori_loop(a, bnd, row_body, zero_accs)
        for k in range(_KV):
            v = jnp.negative(accs[k]) if negate else accs[k]
            plsc.addupdate(acc.at[pl.ds(acc_off + k * _L, _L)], v)

    def group_body(g, carry):
        for i in range(_NBUF):
            c = g * _NBUF + i
            base = c * _CH
            pltpu.make_async_copy(
                hid.at[b, pl.ds(0, _CH), pl.ds(h * _HALF, _HALF)],
                bufs[i], sems[i]).wait()

            a1 = jnp.clip(lo1 - base, 0, _CH)
            b1 = jnp.clip(hi1 - base, 0, _CH)
            a2 = jnp.clip(lo2 - base, 0, _CH)
            b2 = jnp.clip(hi2 - base, 0, _CH)
            seg_accumulate(bufs[i], a1, b1, 0)
            seg_accumulate(bufs[i], a2, b2, _HALF)

            nxt = c + _NBUF

            @pl.when(nxt < _NCH_HEAD)
            def _():
                pltpu.async_copy(
                    hid.at[b, pl.ds(nxt * _CH, _CH), pl.ds(h * _HALF, _HALF)],
                    bufs[i], sems[i])
        return carry

    lax.fori_loop(0, _NCH_HEAD // _NBUF, group_body, 0)

    u1 = hi1 > _R0
    u2 = hi2 > _R0
    b1_hi = jnp.clip(lo1, _R0, _S)
    b2_hi = jnp.clip(lo2, _R0, _S)
    c1_lo = jnp.where(u1, jnp.clip(hi1, _R0, _S), _S)
    c2_lo = jnp.where(u2, jnp.clip(hi2, _R0, _S), _S)

    def range_subtract(lo, hi, acc_off, bufref, sem):
        c0 = lax.shift_right_logical(lo, 6)
        c1 = lax.shift_right_logical(hi + (_CH - 1), 6)

        def chunk_body(ci, carry):
            start = ci * _CH
            pltpu.async_copy(
                hid.at[b, pl.ds(start, _CH), pl.ds(h * _HALF, _HALF)],
                bufref, sem).wait()
            a = jnp.clip(lo - start, 0, _CH)
            e = jnp.clip(hi - start, 0, _CH)
            seg_accumulate(bufref, a, e, acc_off, negate=True)
            return carry

        lax.fori_loop(c0, c1, chunk_body, 0)

    range_subtract(jnp.int32(_R0), b1_hi, 0, bufs[0], sems[0])
    range_subtract(c1_lo, jnp.int32(_S), 0, bufs[0], sems[0])
    range_subtract(jnp.int32(_R0), b2_hi, _HALF, bufs[1], sems[1])
    range_subtract(c2_lo, jnp.int32(_S), _HALF, bufs[1], sems[1])

    def _recip(x):
        xi = lax.bitcast_convert_type(x, jnp.int32)
        y = lax.bitcast_convert_type(jnp.int32(0x7EF311C3) - xi, jnp.float32)
        for _ in range(3):
            y = y * (2.0 - x * y)
        return y

    cnt1 = jnp.maximum(hi1 - lo1, 0)
    cnt2 = jnp.maximum(hi2 - lo2, 0)
    inv1 = _recip(jnp.maximum(cnt1, 1).astype(jnp.float32))
    inv2 = _recip(jnp.maximum(cnt2, 1).astype(jnp.float32))
    inv1v = jnp.full((_L,), inv1, jnp.float32)
    inv2v = jnp.full((_L,), inv2, jnp.float32)
    for k in range(_KV):
        res1[pl.ds(k * _L, _L)] = acc[pl.ds(k * _L, _L)] * inv1v
        res2[pl.ds(k * _L, _L)] = acc[pl.ds(_HALF + k * _L, _L)] * inv2v
    pltpu.sync_copy(res1, p1_out.at[b, pl.ds(h * _HALF, _HALF)])
    pltpu.sync_copy(res2, p2_out.at[b, pl.ds(h * _HALF, _HALF)])

    @pl.when(h == 0)
    def _():
        q1 = jnp.where(u1, inv1, 0.0)
        q2 = jnp.where(u2, inv2, 0.0)
        qv[pl.ds(0, _L)] = jnp.full((_L,), q1, jnp.float32)
        qv[pl.ds(_L, _L)] = jnp.full((_L,), q2, jnp.float32)
        pltpu.sync_copy(qv, q_out.at[b])


def _sc_call(hid, ids, am):
    f = pl.kernel(
        _sc_body,
        out_type=(jax.ShapeDtypeStruct((_B, _D), jnp.float32),
                  jax.ShapeDtypeStruct((_B, _D), jnp.float32),
                  jax.ShapeDtypeStruct((_B, 2 * _L), jnp.float32)),
        mesh=plsc.VectorSubcoreMesh(core_axis_name="c", subcore_axis_name="s",
                                    num_cores=2, num_subcores=16),
        scratch_types=[
            pltpu.VMEM((_S,), jnp.int32),
            pltpu.VMEM((_S,), jnp.int32),
            pltpu.VMEM((2 * _HALF,), jnp.float32),
            pltpu.VMEM((_HALF,), jnp.float32),
            pltpu.VMEM((_HALF,), jnp.float32),
            pltpu.VMEM((2 * _L,), jnp.float32),
            pltpu.VMEM((_CH, _HALF), jnp.float32),
            pltpu.VMEM((_CH, _HALF), jnp.float32),
            pltpu.SemaphoreType.DMA,
            pltpu.SemaphoreType.DMA,
        ],
    )
    return f(hid, ids, am)


def _tc_sum_body(*refs):
    x_refs, o_ref = refs[:-1], refs[-1]
    for i, x_ref in enumerate(x_refs):
        s = jnp.sum(x_ref[...], axis=1)
        o_ref[..., i * _CSW:(i + 1) * _CSW] = s.reshape(1, 1, _BB, _CSW)


def _tc_sum(hid):
    out = pl.pallas_call(
        _tc_sum_body,
        grid=(_B // _BB, _NTB),
        in_specs=[pl.BlockSpec((_BB, _BS, _CSW),
                               lambda b, j, i=i: (b, _R0 // _BS + j, i))
                  for i in range(_NCS)],
        out_specs=pl.BlockSpec((1, 1, _BB, _D), lambda b, j: (b, j, 0, 0)),
        out_shape=jax.ShapeDtypeStruct((_B // _BB, _NTB, _BB, _D),
                                       jnp.float32),
        compiler_params=pltpu.CompilerParams(
            dimension_semantics=("parallel", "arbitrary")),
    )(*([hid] * _NCS))
    return out


def _combine_body(t_ref, p1_ref, p2_ref, q_ref, o1_ref, o2_ref):
    t = jnp.sum(t_ref[...], axis=1).reshape(_B, _D)
    q1 = q_ref[:, 0:1]
    q2 = q_ref[:, _L:_L + 1]
    o1_ref[...] = p1_ref[...] + q1 * t
    o2_ref[...] = p2_ref[...] + q2 * t


def _combine(t, p1, p2, q):
    return pl.pallas_call(
        _combine_body,
        out_shape=(jax.ShapeDtypeStruct((_B, _D), jnp.float32),
                   jax.ShapeDtypeStruct((_B, _D), jnp.float32)),
    )(t, p1, p2, q)


@jax.jit
def kernel(hidden_states, input_ids, attention_mask):
    t = _tc_sum(hidden_states)
    p1, p2, q = _sc_call(hidden_states,
                         input_ids.astype(jnp.int32),
                         attention_mask.astype(jnp.int32))
    return _combine(t, p1, p2, q)
